# baseline (speedup 1.0000x reference)
import jax
import jax.numpy as jnp
from jax.experimental import pallas as pl
from jax.experimental.pallas import tpu as pltpu

N, F, H1, H2, C = 10000, 256, 256, 256, 64

BM = 400
NBLK = N // BM
XCHUNK = 2000


def _gcn_kernel(
    x_hbm_ref,
    adj_ref,
    w1_ref,
    b1_ref,
    w2_ref,
    b2_ref,
    wf_ref,
    bf_ref,
    o_ref,
    x_ref,
    s1_ref,
    s2_ref,
    sem,
):
    i = pl.program_id(0)

    @pl.when(i == 0)
    def _prologue():
        w1 = w1_ref[...].astype(jnp.bfloat16)
        for c in range(N // XCHUNK):
            copy = pltpu.make_async_copy(
                x_hbm_ref.at[pl.ds(c * XCHUNK, XCHUNK), :], x_ref, sem
            )
            copy.start()
            copy.wait()
            s1 = jnp.dot(
                x_ref[...].astype(jnp.bfloat16),
                w1,
                preferred_element_type=jnp.float32,
            )
            s1_ref[pl.ds(c * XCHUNK, XCHUNK), :] = s1.astype(jnp.bfloat16)

    a = adj_ref[...].astype(jnp.bfloat16)

    @pl.when(i < NBLK)
    def _phase_a():
        h = jnp.dot(a, s1_ref[...], preferred_element_type=jnp.float32)
        h = jnp.maximum(h + b1_ref[...], 0.0)
        s2 = jnp.dot(
            h.astype(jnp.bfloat16), w2_ref[...], preferred_element_type=jnp.float32
        )
        s2_ref[pl.ds(i * BM, BM), :] = s2.astype(jnp.bfloat16)

    @pl.when(i >= NBLK)
    def _phase_b():
        h = jnp.dot(a, s2_ref[...], preferred_element_type=jnp.float32)
        h = jnp.maximum(h + b2_ref[...], 0.0)
        logits = (
            jnp.dot(
                h.astype(jnp.bfloat16),
                wf_ref[...],
                preferred_element_type=jnp.float32,
            )
            + bf_ref[...]
        )
        m = jnp.max(logits, axis=1, keepdims=True)
        s = logits - m
        lse = jnp.log(jnp.sum(jnp.exp(s), axis=1, keepdims=True))
        o_ref[...] = s - lse


def kernel(x, adj, W1, b1, W2, b2, Wf, bf):
    return pl.pallas_call(
        _gcn_kernel,
        grid=(2 * NBLK,),
        in_specs=[
            pl.BlockSpec(memory_space=pl.ANY),
            pl.BlockSpec((BM, N), lambda i: (i % NBLK, 0)),
            pl.BlockSpec((F, H1), lambda i: (0, 0)),
            pl.BlockSpec((1, H1), lambda i: (0, 0)),
            pl.BlockSpec((H1, H2), lambda i: (0, 0)),
            pl.BlockSpec((1, H2), lambda i: (0, 0)),
            pl.BlockSpec((H2, C), lambda i: (0, 0)),
            pl.BlockSpec((1, C), lambda i: (0, 0)),
        ],
        out_specs=pl.BlockSpec((BM, C), lambda i: (jnp.maximum(i - NBLK, 0), 0)),
        out_shape=jax.ShapeDtypeStruct((N, C), jnp.float32),
        scratch_shapes=[
            pltpu.VMEM((XCHUNK, F), jnp.float32),
            pltpu.VMEM((N, H1), jnp.bfloat16),
            pltpu.VMEM((N, H2), jnp.bfloat16),
            pltpu.SemaphoreType.DMA,
        ],
    )(
        x,
        adj,
        W1,
        b1.reshape(1, -1),
        W2,
        b2.reshape(1, -1),
        Wf,
        bf.reshape(1, -1),
    )


# speedup vs baseline: 1.0101x; 1.0101x over previous
import jax
import jax.numpy as jnp
from jax.experimental import pallas as pl
from jax.experimental.pallas import tpu as pltpu

N, F, H1, H2, C = 10000, 256, 256, 256, 64

BM = 400
NBLK = N // BM


def _small_matmul_kernel(x_ref, w_ref, o_ref):
    a = x_ref[...].astype(jnp.bfloat16)
    b = w_ref[...].astype(jnp.bfloat16)
    o_ref[...] = jnp.dot(a, b, preferred_element_type=jnp.float32).astype(
        jnp.bfloat16
    )


def _small_matmul(x, w, bm=1000):
    m, k = x.shape
    _, n = w.shape
    return pl.pallas_call(
        _small_matmul_kernel,
        grid=(m // bm,),
        in_specs=[
            pl.BlockSpec((bm, k), lambda i: (i, 0)),
            pl.BlockSpec((k, n), lambda i: (0, 0)),
        ],
        out_specs=pl.BlockSpec((bm, n), lambda i: (i, 0)),
        out_shape=jax.ShapeDtypeStruct((m, n), jnp.bfloat16),
    )(x, w)


def _merged_kernel(
    adj_ref, sup1_ref, b1_ref, w2_ref, b2_ref, wf_ref, bf_ref, o_ref, s2_ref
):
    i = pl.program_id(0)
    a = adj_ref[...].astype(jnp.bfloat16)

    @pl.when(i < NBLK)
    def _phase_a():
        h = jnp.dot(a, sup1_ref[...], preferred_element_type=jnp.float32)
        h = jnp.maximum(h + b1_ref[...], 0.0)
        s2 = jnp.dot(
            h.astype(jnp.bfloat16), w2_ref[...], preferred_element_type=jnp.float32
        )
        s2_ref[pl.ds(i * BM, BM), :] = s2.astype(jnp.bfloat16)

    @pl.when(i >= NBLK)
    def _phase_b():
        h = jnp.dot(a, s2_ref[...], preferred_element_type=jnp.float32)
        h = jnp.maximum(h + b2_ref[...], 0.0)
        logits = (
            jnp.dot(
                h.astype(jnp.bfloat16),
                wf_ref[...],
                preferred_element_type=jnp.float32,
            )
            + bf_ref[...]
        )
        m = jnp.max(logits, axis=1, keepdims=True)
        s = logits - m
        lse = jnp.log(jnp.sum(jnp.exp(s), axis=1, keepdims=True))
        o_ref[...] = s - lse


def kernel(x, adj, W1, b1, W2, b2, Wf, bf):
    support1 = _small_matmul(x, W1)
    return pl.pallas_call(
        _merged_kernel,
        grid=(2 * NBLK,),
        in_specs=[
            pl.BlockSpec(
                (BM, N),
                lambda i: (jnp.where(i < NBLK, i, (i - 1) % NBLK), 0),
            ),
            pl.BlockSpec((N, H1), lambda i: (0, 0)),
            pl.BlockSpec((1, H1), lambda i: (0, 0)),
            pl.BlockSpec((H1, H2), lambda i: (0, 0)),
            pl.BlockSpec((1, H2), lambda i: (0, 0)),
            pl.BlockSpec((H2, C), lambda i: (0, 0)),
            pl.BlockSpec((1, C), lambda i: (0, 0)),
        ],
        out_specs=pl.BlockSpec(
            (BM, C),
            lambda i: (jnp.where(i < NBLK, NBLK - 1, (i - 1) % NBLK), 0),
        ),
        out_shape=jax.ShapeDtypeStruct((N, C), jnp.float32),
        scratch_shapes=[pltpu.VMEM((N, H2), jnp.bfloat16)],
    )(
        adj,
        support1,
        b1.reshape(1, -1),
        W2,
        b2.reshape(1, -1),
        Wf,
        bf.reshape(1, -1),
    )


# speedup vs baseline: 1.0207x; 1.0104x over previous
import jax
import jax.numpy as jnp
from jax.experimental import pallas as pl
from jax.experimental.pallas import tpu as pltpu

N, F, H1, H2, C = 10000, 256, 256, 256, 64

BM = 400
NBLK = N // BM


def _small_matmul_kernel(x_ref, w_ref, o_ref):
    a = x_ref[...].astype(jnp.bfloat16)
    b = w_ref[...].astype(jnp.bfloat16)
    o_ref[...] = jnp.dot(a, b, preferred_element_type=jnp.float32).astype(
        jnp.bfloat16
    )


def _small_matmul(x, w, bm=10000):
    m, k = x.shape
    _, n = w.shape
    return pl.pallas_call(
        _small_matmul_kernel,
        grid=(m // bm,),
        in_specs=[
            pl.BlockSpec((bm, k), lambda i: (i, 0)),
            pl.BlockSpec((k, n), lambda i: (0, 0)),
        ],
        out_specs=pl.BlockSpec((bm, n), lambda i: (i, 0)),
        out_shape=jax.ShapeDtypeStruct((m, n), jnp.bfloat16),
    )(x, w)


def _merged_kernel(
    adj_ref, sup1_ref, b1_ref, w2_ref, b2_ref, wf_ref, bf_ref, o_ref, s2_ref
):
    i = pl.program_id(0)
    a = adj_ref[...].astype(jnp.bfloat16)

    @pl.when(i < NBLK)
    def _phase_a():
        h = jnp.dot(a, sup1_ref[...], preferred_element_type=jnp.float32)
        h = jnp.maximum(h + b1_ref[...], 0.0)
        s2 = jnp.dot(
            h.astype(jnp.bfloat16), w2_ref[...], preferred_element_type=jnp.float32
        )
        s2_ref[pl.ds(i * BM, BM), :] = s2.astype(jnp.bfloat16)

    @pl.when(i >= NBLK)
    def _phase_b():
        h = jnp.dot(a, s2_ref[...], preferred_element_type=jnp.float32)
        h = jnp.maximum(h + b2_ref[...], 0.0)
        logits = (
            jnp.dot(
                h.astype(jnp.bfloat16),
                wf_ref[...],
                preferred_element_type=jnp.float32,
            )
            + bf_ref[...]
        )
        m = jnp.max(logits, axis=1, keepdims=True)
        s = logits - m
        lse = jnp.log(jnp.sum(jnp.exp(s), axis=1, keepdims=True))
        o_ref[...] = s - lse


def kernel(x, adj, W1, b1, W2, b2, Wf, bf):
    support1 = _small_matmul(x, W1)
    return pl.pallas_call(
        _merged_kernel,
        grid=(2 * NBLK,),
        in_specs=[
            pl.BlockSpec(
                (BM, N),
                lambda i: (jnp.where(i < NBLK, i, (i - 1) % NBLK), 0),
            ),
            pl.BlockSpec((N, H1), lambda i: (0, 0)),
            pl.BlockSpec((1, H1), lambda i: (0, 0)),
            pl.BlockSpec((H1, H2), lambda i: (0, 0)),
            pl.BlockSpec((1, H2), lambda i: (0, 0)),
            pl.BlockSpec((H2, C), lambda i: (0, 0)),
            pl.BlockSpec((1, C), lambda i: (0, 0)),
        ],
        out_specs=pl.BlockSpec(
            (BM, C),
            lambda i: (jnp.where(i < NBLK, NBLK - 1, (i - 1) % NBLK), 0),
        ),
        out_shape=jax.ShapeDtypeStruct((N, C), jnp.float32),
        scratch_shapes=[pltpu.VMEM((N, H2), jnp.bfloat16)],
    )(
        adj,
        support1,
        b1.reshape(1, -1),
        W2,
        b2.reshape(1, -1),
        Wf,
        bf.reshape(1, -1),
    )


# speedup vs baseline: 1.0253x; 1.0045x over previous
import jax
import jax.numpy as jnp
from jax.experimental import pallas as pl
from jax.experimental.pallas import tpu as pltpu

N, F, H1, H2, C = 10000, 256, 256, 256, 64

BM = 400
NBLK = N // BM


def _gcn_kernel(
    adj_ref, x_ref, w1_ref, b1_ref, w2_ref, b2_ref, wf_ref, bf_ref, o_ref, s2_ref
):
    i = pl.program_id(0)
    a = adj_ref[...].astype(jnp.bfloat16)

    @pl.when(i < NBLK)
    def _phase_a():
        hp = jnp.dot(a, x_ref[...], preferred_element_type=jnp.float32)
        h = jnp.dot(
            hp.astype(jnp.bfloat16),
            w1_ref[...].astype(jnp.bfloat16),
            preferred_element_type=jnp.float32,
        )
        h = jnp.maximum(h + b1_ref[...], 0.0)
        s2 = jnp.dot(
            h.astype(jnp.bfloat16), w2_ref[...], preferred_element_type=jnp.float32
        )
        s2_ref[pl.ds(i * BM, BM), :] = s2.astype(jnp.bfloat16)

    @pl.when(i >= NBLK)
    def _phase_b():
        h = jnp.dot(a, s2_ref[...], preferred_element_type=jnp.float32)
        h = jnp.maximum(h + b2_ref[...], 0.0)
        logits = (
            jnp.dot(
                h.astype(jnp.bfloat16),
                wf_ref[...],
                preferred_element_type=jnp.float32,
            )
            + bf_ref[...]
        )
        m = jnp.max(logits, axis=1, keepdims=True)
        s = logits - m
        lse = jnp.log(jnp.sum(jnp.exp(s), axis=1, keepdims=True))
        o_ref[...] = s - lse


def kernel(x, adj, W1, b1, W2, b2, Wf, bf):
    return pl.pallas_call(
        _gcn_kernel,
        grid=(2 * NBLK,),
        in_specs=[
            pl.BlockSpec(
                (BM, N),
                lambda i: (jnp.where(i < NBLK, i, (i - 1) % NBLK), 0),
            ),
            pl.BlockSpec((N, F), lambda i: (0, 0)),
            pl.BlockSpec((F, H1), lambda i: (0, 0)),
            pl.BlockSpec((1, H1), lambda i: (0, 0)),
            pl.BlockSpec((H1, H2), lambda i: (0, 0)),
            pl.BlockSpec((1, H2), lambda i: (0, 0)),
            pl.BlockSpec((H2, C), lambda i: (0, 0)),
            pl.BlockSpec((1, C), lambda i: (0, 0)),
        ],
        out_specs=pl.BlockSpec(
            (BM, C),
            lambda i: (jnp.where(i < NBLK, NBLK - 1, (i - 1) % NBLK), 0),
        ),
        out_shape=jax.ShapeDtypeStruct((N, C), jnp.float32),
        scratch_shapes=[pltpu.VMEM((N, H2), jnp.bfloat16)],
    )(
        adj,
        x.astype(jnp.bfloat16),
        W1,
        b1.reshape(1, -1),
        W2,
        b2.reshape(1, -1),
        Wf,
        bf.reshape(1, -1),
    )


# speedup vs baseline: 1.0566x; 1.0306x over previous
import jax
import jax.numpy as jnp
from jax.experimental import pallas as pl
from jax.experimental.pallas import tpu as pltpu

N, F, H1, H2, C = 10000, 256, 256, 256, 64

BM = 400
NBLK = N // BM


def _gcn_kernel(
    adj_ref, x_ref, w1_ref, b1_ref, w2_ref, b2_ref, wf_ref, bf_ref, o_ref, s2_ref
):
    i = pl.program_id(0)

    @pl.when(i < NBLK)
    def _phase_a():
        hp = jnp.dot(
            adj_ref[...], x_ref[...], preferred_element_type=jnp.float32
        )
        h = jnp.dot(
            hp.astype(jnp.bfloat16),
            w1_ref[...].astype(jnp.bfloat16),
            preferred_element_type=jnp.float32,
        )
        h = jnp.maximum(h + b1_ref[...], 0.0)
        s2 = jnp.dot(
            h.astype(jnp.bfloat16), w2_ref[...], preferred_element_type=jnp.float32
        )
        s2_ref[pl.ds(i * BM, BM), :] = s2.astype(jnp.bfloat16)

    @pl.when(i >= NBLK)
    def _phase_b():
        a = adj_ref[...].astype(jnp.bfloat16)
        h = jnp.dot(a, s2_ref[...], preferred_element_type=jnp.float32)
        h = jnp.maximum(h + b2_ref[...], 0.0)
        logits = (
            jnp.dot(
                h.astype(jnp.bfloat16),
                wf_ref[...],
                preferred_element_type=jnp.float32,
            )
            + bf_ref[...]
        )
        m = jnp.max(logits, axis=1, keepdims=True)
        s = logits - m
        lse = jnp.log(jnp.sum(jnp.exp(s), axis=1, keepdims=True))
        o_ref[...] = s - lse


def kernel(x, adj, W1, b1, W2, b2, Wf, bf):
    return pl.pallas_call(
        _gcn_kernel,
        grid=(2 * NBLK,),
        in_specs=[
            pl.BlockSpec(
                (BM, N),
                lambda i: (jnp.where(i < NBLK, i, (i - 1) % NBLK), 0),
            ),
            pl.BlockSpec((N, F), lambda i: (0, 0)),
            pl.BlockSpec((F, H1), lambda i: (0, 0)),
            pl.BlockSpec((1, H1), lambda i: (0, 0)),
            pl.BlockSpec((H1, H2), lambda i: (0, 0)),
            pl.BlockSpec((1, H2), lambda i: (0, 0)),
            pl.BlockSpec((H2, C), lambda i: (0, 0)),
            pl.BlockSpec((1, C), lambda i: (0, 0)),
        ],
        out_specs=pl.BlockSpec(
            (BM, C),
            lambda i: (jnp.where(i < NBLK, NBLK - 1, (i - 1) % NBLK), 0),
        ),
        out_shape=jax.ShapeDtypeStruct((N, C), jnp.float32),
        scratch_shapes=[pltpu.VMEM((N, H2), jnp.bfloat16)],
    )(
        adj,
        x,
        W1,
        b1.reshape(1, -1),
        W2,
        b2.reshape(1, -1),
        Wf,
        bf.reshape(1, -1),
    )
